# baseline (speedup 1.0000x reference)
import jax
import jax.numpy as jnp
from jax import lax
from jax.experimental import pallas as pl
from jax.experimental.pallas import tpu as pltpu
from jax.experimental.pallas import tpu_sc as plsc

N_NODES = 10000
N_EDGES = 320000
D_IN = 128
D_HID = 16
D_OUT = 128

NTILES = 16
NPT = N_NODES // NTILES
CH = 125
NCHUNKS_TOTAL = N_EDGES // CH
NCH = NCHUNKS_TOTAL // NTILES
EPS = 1e-5


ROWS_BLK = 1000
GRID_ROWS = N_NODES // ROWS_BLK


def _gelu(x):
    return 0.5 * x * (1.0 + lax.erf(x * (2.0 ** -0.5)))


def _tc1_body(x_ref, w1_ref, b1_ref, g1_ref, be1_ref, h_ref):
    x = x_ref[...]
    h = lax.dot_general(x, w1_ref[...], (((1,), (1,)), ((), ())),
                        preferred_element_type=jnp.float32)
    h = _gelu(h + b1_ref[...])
    mu = jnp.mean(h, axis=-1, keepdims=True)
    var = jnp.mean((h - mu) ** 2, axis=-1, keepdims=True)
    h_ref[...] = (h - mu) / jnp.sqrt(var + EPS) * g1_ref[...] + be1_ref[...]


def _tc1(x, W1, b1, g1, be1):
    return pl.pallas_call(
        _tc1_body,
        out_shape=jax.ShapeDtypeStruct((N_NODES, D_HID), jnp.float32),
        grid=(GRID_ROWS,),
        in_specs=[
            pl.BlockSpec((ROWS_BLK, D_IN), lambda i: (i, 0)),
            pl.BlockSpec((D_HID, D_IN), lambda i: (0, 0)),
            pl.BlockSpec((1, D_HID), lambda i: (0, 0)),
            pl.BlockSpec((1, D_HID), lambda i: (0, 0)),
            pl.BlockSpec((1, D_HID), lambda i: (0, 0)),
        ],
        out_specs=pl.BlockSpec((ROWS_BLK, D_HID), lambda i: (i, 0)),
    )(x, W1, b1.reshape(1, D_HID), g1.reshape(1, D_HID), be1.reshape(1, D_HID))


def _rsqrt16(x):
    i = plsc.bitcast(x, jnp.int32)
    i = jnp.int32(0x5F3759DF) - (i >> 1)
    y = plsc.bitcast(i, jnp.float32)
    for _ in range(3):
        y = y * (1.5 - 0.5 * x * y * y)
    return y


def _sc_body(rows_hbm, cols_hbm, h_hbm, c6_hbm, c8_hbm, c10_hbm, u_hbm,
             s_sh, rowix, colix, gbuf0, gbuf1, gbuf2, gbuf3, onesb,
             nbufS, nbufU, d2b, sdb, zbuf,
             gsem0, gsem1, gsem2, gsem3, ssem0, ssem1, ssem2, ssem3):
    cid = lax.axis_index("c")
    tid = lax.axis_index("s")

    @pl.when(cid == 0)
    def _work():
        nbase = tid * NPT
        cbase = tid * NCH

        pltpu.sync_copy(rows_hbm.at[pl.ds(cbase, NCH), :], rowix)
        pltpu.sync_copy(cols_hbm.at[pl.ds(cbase, NCH), :], colix)

        gbufs = (gbuf0, gbuf1, gbuf2, gbuf3)
        gsems = (gsem0, gsem1, gsem2, gsem3)
        ssems = (ssem0, ssem1, ssem2, ssem3)

        def g_fire(j, b):
            pltpu.async_copy(u_hbm.at[rowix.at[j]], gbufs[b], gsems[b])

        def g_wait(j, b):
            pltpu.make_async_copy(u_hbm.at[rowix.at[j]], gbufs[b],
                                  gsems[b]).wait()

        def s_fire(j, b, src=None):
            pltpu.async_copy(src if src is not None else gbufs[b],
                             s_sh.at[colix.at[j]], ssems[b], add=True)

        def s_wait(j, b, src=None):
            pltpu.make_async_copy(src if src is not None else gbufs[b],
                                  s_sh.at[colix.at[j]], ssems[b]).wait()

        def _fill_const(i, c):
            zbuf[i, :] = jnp.zeros((D_HID,), jnp.float32)
            return c
        lax.fori_loop(0, NPT, _fill_const, 0)

        def _fill_ones(i, c):
            onesb[i, :] = jnp.ones((D_HID,), jnp.float32)
            return c
        lax.fori_loop(0, CH, _fill_ones, 0)

        pltpu.sync_copy(zbuf, s_sh.at[pl.ds(nbase, NPT), :])
        plsc.subcore_barrier()

        for b in range(4):
            s_fire(b, b, src=onesb)

        def _hist(jj, c):
            for b in range(4):
                j = 4 * jj + 4 + b
                s_wait(j - 4, b, src=onesb)
                s_fire(j, b, src=onesb)
            return c
        lax.fori_loop(0, (NCH - 4) // 4, _hist, 0)
        for b in range(4):
            s_wait(NCH - 4 + b, b, src=onesb)
        plsc.subcore_barrier()

        pltpu.sync_copy(s_sh.at[pl.ds(nbase, NPT), :], nbufS)
        pltpu.sync_copy(h_hbm.at[pl.ds(nbase, NPT), :], nbufU)

        def _setup(i, c):
            deg = nbufS[i, :] + 1.0
            r = _rsqrt16(deg)
            d2b[i, :] = 1.0 / deg
            sdb[i, :] = deg * r
            nbufU[i, :] = r * nbufU[i, :]
            return c
        lax.fori_loop(0, NPT, _setup, 0)

        pltpu.sync_copy(nbufU, u_hbm.at[pl.ds(nbase, NPT), :])
        pltpu.sync_copy(zbuf, s_sh.at[pl.ds(nbase, NPT), :])
        plsc.subcore_barrier()

        taps = {6: c6_hbm, 8: c8_hbm, 10: c10_hbm}
        for p in range(1, 11):
          with jax.named_scope(f"edge_phase_{p}"):
            g_fire(0, 0)
            g_fire(1, 1)
            g_wait(0, 0)
            s_fire(0, 0)
            g_fire(2, 2)
            g_wait(1, 1)
            s_fire(1, 1)
            g_fire(3, 3)

            def _edges(jj, c):
                for k in range(4):
                    j = 4 * jj + 2 + k
                    b = (2 + k) % 4
                    g_wait(j, b)
                    s_fire(j, b)
                    s_wait(j - 2, (b + 2) % 4)
                    g_fire(j + 2, (b + 2) % 4)
                return c
            lax.fori_loop(0, (NCH - 4) // 4, _edges, 0)
            g_wait(NCH - 2, 2)
            s_fire(NCH - 2, 2)
            s_wait(NCH - 4, 0)
            g_wait(NCH - 1, 3)
            s_fire(NCH - 1, 3)
            s_wait(NCH - 3, 1)
            s_wait(NCH - 2, 2)
            s_wait(NCH - 1, 3)
            plsc.subcore_barrier()

          with jax.named_scope(f"node_phase_{p}"):
            pltpu.sync_copy(s_sh.at[pl.ds(nbase, NPT), :], nbufS)

            def _update(i, c):
                nbufU[i, :] = d2b[i, :] * (nbufS[i, :] + nbufU[i, :])
                return c
            lax.fori_loop(0, NPT, _update, 0)

            pltpu.sync_copy(nbufU, u_hbm.at[pl.ds(nbase, NPT), :])
            pltpu.sync_copy(zbuf, s_sh.at[pl.ds(nbase, NPT), :])

            if p in taps:
                def _tap(i, c):
                    nbufS[i, :] = sdb[i, :] * nbufU[i, :]
                    return c
                lax.fori_loop(0, NPT, _tap, 0)
                pltpu.sync_copy(nbufS, taps[p].at[pl.ds(nbase, NPT), :])
            plsc.subcore_barrier()


def _sc_prop(rows_r, cols_r, h):
    mesh = plsc.VectorSubcoreMesh(core_axis_name="c", subcore_axis_name="s")
    f = pl.kernel(
        _sc_body,
        out_type=(
            jax.ShapeDtypeStruct((N_NODES, D_HID), jnp.float32),
            jax.ShapeDtypeStruct((N_NODES, D_HID), jnp.float32),
            jax.ShapeDtypeStruct((N_NODES, D_HID), jnp.float32),
            jax.ShapeDtypeStruct((N_NODES, D_HID), jnp.float32),
        ),
        mesh=mesh,
        compiler_params=pltpu.CompilerParams(use_tc_tiling_on_sc=False,
                                              needs_layout_passes=False),
        scratch_types=[
            pltpu.VMEM_SHARED((N_NODES, D_HID), jnp.float32),
            pltpu.VMEM((NCH, CH), jnp.int32),
            pltpu.VMEM((NCH, CH), jnp.int32),
            pltpu.VMEM((CH, D_HID), jnp.float32),
            pltpu.VMEM((CH, D_HID), jnp.float32),
            pltpu.VMEM((CH, D_HID), jnp.float32),
            pltpu.VMEM((CH, D_HID), jnp.float32),
            pltpu.VMEM((CH, D_HID), jnp.float32),
            pltpu.VMEM((NPT, D_HID), jnp.float32),
            pltpu.VMEM((NPT, D_HID), jnp.float32),
            pltpu.VMEM((NPT, D_HID), jnp.float32),
            pltpu.VMEM((NPT, D_HID), jnp.float32),
            pltpu.VMEM((NPT, D_HID), jnp.float32),
        ] + [pltpu.SemaphoreType.DMA] * 8,
    )
    return f(rows_r, cols_r, h)[:3]


D_CAT = 3 * D_HID


def _tc2_body(c6_ref, c8_ref, c10_ref, w6_ref, b6_ref, w8_ref, b8_ref,
              w10_ref, b10_ref, g2_ref, be2_ref, w2_ref, b2_ref, out_ref):
    def lin(c_ref, w_ref, b_ref):
        return lax.dot_general(c_ref[...], w_ref[...], (((1,), (1,)), ((), ())),
                               preferred_element_type=jnp.float32) + b_ref[...]

    t6 = _gelu(lin(c6_ref, w6_ref, b6_ref))
    t8 = _gelu(lin(c8_ref, w8_ref, b8_ref))
    t10 = _gelu(lin(c10_ref, w10_ref, b10_ref))

    s = jnp.sum(t6, axis=-1, keepdims=True) + jnp.sum(t8, axis=-1, keepdims=True) \
        + jnp.sum(t10, axis=-1, keepdims=True)
    mu = s / D_CAT
    v = (jnp.sum((t6 - mu) ** 2, axis=-1, keepdims=True)
         + jnp.sum((t8 - mu) ** 2, axis=-1, keepdims=True)
         + jnp.sum((t10 - mu) ** 2, axis=-1, keepdims=True)) / D_CAT
    inv = 1.0 / jnp.sqrt(v + EPS)

    g2 = g2_ref[...]
    be2 = be2_ref[...]
    w2 = w2_ref[...]
    acc = jnp.zeros_like(out_ref[...]) + b2_ref[...]
    for k, t in enumerate((t6, t8, t10)):
        nk = (t - mu) * inv * g2[:, k * D_HID:(k + 1) * D_HID] \
            + be2[:, k * D_HID:(k + 1) * D_HID]
        acc = acc + lax.dot_general(
            nk, w2[:, k * D_HID:(k + 1) * D_HID], (((1,), (1,)), ((), ())),
            preferred_element_type=jnp.float32)
    out_ref[...] = acc


def _tc2(c6, c8, c10, W6, b6, W8, b8, W10, b10, g2, be2, W2, b2):
    blk16 = pl.BlockSpec((ROWS_BLK, D_HID), lambda i: (i, 0))
    w16 = pl.BlockSpec((D_HID, D_HID), lambda i: (0, 0))
    v16 = pl.BlockSpec((1, D_HID), lambda i: (0, 0))
    v48 = pl.BlockSpec((1, D_CAT), lambda i: (0, 0))
    return pl.pallas_call(
        _tc2_body,
        out_shape=jax.ShapeDtypeStruct((N_NODES, D_OUT), jnp.float32),
        grid=(GRID_ROWS,),
        in_specs=[
            blk16, blk16, blk16,
            w16, v16, w16, v16, w16, v16,
            v48, v48,
            pl.BlockSpec((D_OUT, D_CAT), lambda i: (0, 0)),
            pl.BlockSpec((1, D_OUT), lambda i: (0, 0)),
        ],
        out_specs=pl.BlockSpec((ROWS_BLK, D_OUT), lambda i: (i, 0)),
    )(c6, c8, c10,
      W6, b6.reshape(1, D_HID), W8, b8.reshape(1, D_HID),
      W10, b10.reshape(1, D_HID),
      g2.reshape(1, D_CAT), be2.reshape(1, D_CAT),
      W2, b2.reshape(1, D_OUT))


def kernel(x, edge_index, W1, b1, W6, b6, W8, b8, W10, b10,
           g1, be1, g2, be2, W2, b2):
    h = _tc1(x, W1, b1, g1, be1)
    rows_r = edge_index[0].reshape(NCHUNKS_TOTAL, CH)
    cols_r = edge_index[1].reshape(NCHUNKS_TOTAL, CH)
    c6, c8, c10 = _sc_prop(rows_r, cols_r, h)
    return _tc2(c6, c8, c10, W6, b6, W8, b8, W10, b10, g2, be2, W2, b2)


# speedup vs baseline: 1.7113x; 1.7113x over previous
import jax
import jax.numpy as jnp
from jax import lax
from jax.experimental import pallas as pl
from jax.experimental.pallas import tpu as pltpu
from jax.experimental.pallas import tpu_sc as plsc

N_NODES = 10000
N_EDGES = 320000
D_IN = 128
D_HID = 16
D_OUT = 128

NTILES = 16
NPT = N_NODES // NTILES
CH = 125
NCHUNKS_TOTAL = N_EDGES // CH
NCH = NCHUNKS_TOTAL // NTILES
EPS = 1e-5


ROWS_BLK = 1000
GRID_ROWS = N_NODES // ROWS_BLK


def _gelu(x):
    return 0.5 * x * (1.0 + lax.erf(x * (2.0 ** -0.5)))


def _tc1_body(x_ref, w1_ref, b1_ref, g1_ref, be1_ref, h_ref):
    x = x_ref[...]
    h = lax.dot_general(x, w1_ref[...], (((1,), (1,)), ((), ())),
                        preferred_element_type=jnp.float32)
    h = _gelu(h + b1_ref[...])
    mu = jnp.mean(h, axis=-1, keepdims=True)
    var = jnp.mean((h - mu) ** 2, axis=-1, keepdims=True)
    h_ref[...] = (h - mu) / jnp.sqrt(var + EPS) * g1_ref[...] + be1_ref[...]


def _tc1(x, W1, b1, g1, be1):
    return pl.pallas_call(
        _tc1_body,
        out_shape=jax.ShapeDtypeStruct((N_NODES, D_HID), jnp.float32),
        grid=(GRID_ROWS,),
        in_specs=[
            pl.BlockSpec((ROWS_BLK, D_IN), lambda i: (i, 0)),
            pl.BlockSpec((D_HID, D_IN), lambda i: (0, 0)),
            pl.BlockSpec((1, D_HID), lambda i: (0, 0)),
            pl.BlockSpec((1, D_HID), lambda i: (0, 0)),
            pl.BlockSpec((1, D_HID), lambda i: (0, 0)),
        ],
        out_specs=pl.BlockSpec((ROWS_BLK, D_HID), lambda i: (i, 0)),
    )(x, W1, b1.reshape(1, D_HID), g1.reshape(1, D_HID), be1.reshape(1, D_HID))


def _rsqrt16(x):
    i = plsc.bitcast(x, jnp.int32)
    i = jnp.int32(0x5F3759DF) - (i >> 1)
    y = plsc.bitcast(i, jnp.float32)
    for _ in range(3):
        y = y * (1.5 - 0.5 * x * y * y)
    return y


def _sc_body(rows_hbm, cols_hbm, h_hbm, taps_hbm,
             u_sh, s_sh, rowix, colix, gbuf0, gbuf1, gbuf2, gbuf3, onesb,
             nbufS, nbufU, d2b, sdb, zbuf,
             gsem0, gsem1, gsem2, gsem3, ssem0, ssem1, ssem2, ssem3):
    cid = lax.axis_index("c")
    tid = lax.axis_index("s")

    @pl.when(cid == 0)
    def _work():
        nbase = tid * NPT
        cbase = tid * NCH

        pltpu.sync_copy(rows_hbm.at[pl.ds(cbase, NCH), :], rowix)
        pltpu.sync_copy(cols_hbm.at[pl.ds(cbase, NCH), :], colix)

        gbufs = (gbuf0, gbuf1, gbuf2, gbuf3)
        gsems = (gsem0, gsem1, gsem2, gsem3)
        ssems = (ssem0, ssem1, ssem2, ssem3)

        def g_fire(j, b):
            pltpu.async_copy(u_sh.at[rowix.at[j]], gbufs[b], gsems[b])

        def g_wait(j, b):
            pltpu.make_async_copy(u_sh.at[rowix.at[j]], gbufs[b],
                                  gsems[b]).wait()

        def s_fire(j, b, src=None):
            pltpu.async_copy(src if src is not None else gbufs[b],
                             s_sh.at[colix.at[j]], ssems[b], add=True)

        def s_wait(j, b, src=None):
            pltpu.make_async_copy(src if src is not None else gbufs[b],
                                  s_sh.at[colix.at[j]], ssems[b]).wait()

        def _fill_const(i, c):
            zbuf[i, :] = jnp.zeros((D_HID,), jnp.float32)
            return c
        lax.fori_loop(0, NPT, _fill_const, 0)

        def _fill_ones(i, c):
            onesb[i, :] = jnp.ones((D_HID,), jnp.float32)
            return c
        lax.fori_loop(0, CH, _fill_ones, 0)

        pltpu.sync_copy(zbuf, s_sh.at[pl.ds(nbase, NPT), :])
        plsc.subcore_barrier()

        for b in range(4):
            s_fire(b, b, src=onesb)

        def _hist(jj, c):
            for b in range(4):
                j = 4 * jj + 4 + b
                s_wait(j - 4, b, src=onesb)
                s_fire(j, b, src=onesb)
            return c
        lax.fori_loop(0, (NCH - 4) // 4, _hist, 0)
        for b in range(4):
            s_wait(NCH - 4 + b, b, src=onesb)
        plsc.subcore_barrier()

        pltpu.sync_copy(s_sh.at[pl.ds(nbase, NPT), :], nbufS)
        pltpu.sync_copy(h_hbm.at[pl.ds(nbase, NPT), :], nbufU)

        def _setup(i, c):
            deg = nbufS[i, :] + 1.0
            r = _rsqrt16(deg)
            d2b[i, :] = 1.0 / deg
            sdb[i, :] = deg * r
            nbufU[i, :] = r * nbufU[i, :]
            return c
        lax.fori_loop(0, NPT, _setup, 0)

        pltpu.sync_copy(nbufU, u_sh.at[pl.ds(nbase, NPT), :])
        pltpu.sync_copy(zbuf, s_sh.at[pl.ds(nbase, NPT), :])
        plsc.subcore_barrier()

        def _round(p, carry):
            g_fire(0, 0)
            g_fire(1, 1)
            g_wait(0, 0)
            s_fire(0, 0)
            g_fire(2, 2)
            g_wait(1, 1)
            s_fire(1, 1)
            g_fire(3, 3)

            def _edges(jj, c):
                for k in range(4):
                    j = 4 * jj + 2 + k
                    b = (2 + k) % 4
                    g_wait(j, b)
                    s_fire(j, b)
                    s_wait(j - 2, (b + 2) % 4)
                    g_fire(j + 2, (b + 2) % 4)
                return c
            lax.fori_loop(0, (NCH - 4) // 4, _edges, 0)
            g_wait(NCH - 2, 2)
            s_fire(NCH - 2, 2)
            s_wait(NCH - 4, 0)
            g_wait(NCH - 1, 3)
            s_fire(NCH - 1, 3)
            s_wait(NCH - 3, 1)
            s_wait(NCH - 2, 2)
            s_wait(NCH - 1, 3)
            plsc.subcore_barrier()

            pltpu.sync_copy(s_sh.at[pl.ds(nbase, NPT), :], nbufS)

            def _update(i, c):
                nbufU[i, :] = d2b[i, :] * (nbufS[i, :] + nbufU[i, :])
                return c
            lax.fori_loop(0, NPT, _update, 0)

            pltpu.sync_copy(nbufU, u_sh.at[pl.ds(nbase, NPT), :])
            pltpu.sync_copy(zbuf, s_sh.at[pl.ds(nbase, NPT), :])

            @pl.when(jnp.logical_and(p >= 6, lax.rem(p, 2) == 0))
            def _():
                def _tap(i, c):
                    nbufS[i, :] = sdb[i, :] * nbufU[i, :]
                    return c
                lax.fori_loop(0, NPT, _tap, 0)
                t = lax.div(p - 6, 2)
                pltpu.sync_copy(nbufS, taps_hbm.at[t, pl.ds(nbase, NPT), :])
            plsc.subcore_barrier()
            return carry
        lax.fori_loop(1, 11, _round, 0)


def _sc_prop(rows_r, cols_r, h):
    mesh = plsc.VectorSubcoreMesh(core_axis_name="c", subcore_axis_name="s")
    f = pl.kernel(
        _sc_body,
        out_type=jax.ShapeDtypeStruct((3, N_NODES, D_HID), jnp.float32),
        mesh=mesh,
        compiler_params=pltpu.CompilerParams(use_tc_tiling_on_sc=False,
                                              needs_layout_passes=False),
        scratch_types=[
            pltpu.VMEM_SHARED((N_NODES, D_HID), jnp.float32),
            pltpu.VMEM_SHARED((N_NODES, D_HID), jnp.float32),
            pltpu.VMEM((NCH, CH), jnp.int32),
            pltpu.VMEM((NCH, CH), jnp.int32),
            pltpu.VMEM((CH, D_HID), jnp.float32),
            pltpu.VMEM((CH, D_HID), jnp.float32),
            pltpu.VMEM((CH, D_HID), jnp.float32),
            pltpu.VMEM((CH, D_HID), jnp.float32),
            pltpu.VMEM((CH, D_HID), jnp.float32),
            pltpu.VMEM((NPT, D_HID), jnp.float32),
            pltpu.VMEM((NPT, D_HID), jnp.float32),
            pltpu.VMEM((NPT, D_HID), jnp.float32),
            pltpu.VMEM((NPT, D_HID), jnp.float32),
            pltpu.VMEM((NPT, D_HID), jnp.float32),
        ] + [pltpu.SemaphoreType.DMA] * 8,
    )
    taps = f(rows_r, cols_r, h)
    return taps[0], taps[1], taps[2]


D_CAT = 3 * D_HID


def _tc2_body(c6_ref, c8_ref, c10_ref, w6_ref, b6_ref, w8_ref, b8_ref,
              w10_ref, b10_ref, g2_ref, be2_ref, w2_ref, b2_ref, out_ref):
    def lin(c_ref, w_ref, b_ref):
        return lax.dot_general(c_ref[...], w_ref[...], (((1,), (1,)), ((), ())),
                               preferred_element_type=jnp.float32) + b_ref[...]

    t6 = _gelu(lin(c6_ref, w6_ref, b6_ref))
    t8 = _gelu(lin(c8_ref, w8_ref, b8_ref))
    t10 = _gelu(lin(c10_ref, w10_ref, b10_ref))

    s = jnp.sum(t6, axis=-1, keepdims=True) + jnp.sum(t8, axis=-1, keepdims=True) \
        + jnp.sum(t10, axis=-1, keepdims=True)
    mu = s / D_CAT
    v = (jnp.sum((t6 - mu) ** 2, axis=-1, keepdims=True)
         + jnp.sum((t8 - mu) ** 2, axis=-1, keepdims=True)
         + jnp.sum((t10 - mu) ** 2, axis=-1, keepdims=True)) / D_CAT
    inv = 1.0 / jnp.sqrt(v + EPS)

    g2 = g2_ref[...]
    be2 = be2_ref[...]
    w2 = w2_ref[...]
    acc = jnp.zeros_like(out_ref[...]) + b2_ref[...]
    for k, t in enumerate((t6, t8, t10)):
        nk = (t - mu) * inv * g2[:, k * D_HID:(k + 1) * D_HID] \
            + be2[:, k * D_HID:(k + 1) * D_HID]
        acc = acc + lax.dot_general(
            nk, w2[:, k * D_HID:(k + 1) * D_HID], (((1,), (1,)), ((), ())),
            preferred_element_type=jnp.float32)
    out_ref[...] = acc


def _tc2(c6, c8, c10, W6, b6, W8, b8, W10, b10, g2, be2, W2, b2):
    blk16 = pl.BlockSpec((ROWS_BLK, D_HID), lambda i: (i, 0))
    w16 = pl.BlockSpec((D_HID, D_HID), lambda i: (0, 0))
    v16 = pl.BlockSpec((1, D_HID), lambda i: (0, 0))
    v48 = pl.BlockSpec((1, D_CAT), lambda i: (0, 0))
    return pl.pallas_call(
        _tc2_body,
        out_shape=jax.ShapeDtypeStruct((N_NODES, D_OUT), jnp.float32),
        grid=(GRID_ROWS,),
        in_specs=[
            blk16, blk16, blk16,
            w16, v16, w16, v16, w16, v16,
            v48, v48,
            pl.BlockSpec((D_OUT, D_CAT), lambda i: (0, 0)),
            pl.BlockSpec((1, D_OUT), lambda i: (0, 0)),
        ],
        out_specs=pl.BlockSpec((ROWS_BLK, D_OUT), lambda i: (i, 0)),
    )(c6, c8, c10,
      W6, b6.reshape(1, D_HID), W8, b8.reshape(1, D_HID),
      W10, b10.reshape(1, D_HID),
      g2.reshape(1, D_CAT), be2.reshape(1, D_CAT),
      W2, b2.reshape(1, D_OUT))


def kernel(x, edge_index, W1, b1, W6, b6, W8, b8, W10, b10,
           g1, be1, g2, be2, W2, b2):
    h = _tc1(x, W1, b1, g1, be1)
    rows_r = edge_index[0].reshape(NCHUNKS_TOTAL, CH)
    cols_r = edge_index[1].reshape(NCHUNKS_TOTAL, CH)
    c6, c8, c10 = _sc_prop(rows_r, cols_r, h)
    return _tc2(c6, c8, c10, W6, b6, W8, b8, W10, b10, g2, be2, W2, b2)


# speedup vs baseline: 2.3459x; 1.3709x over previous
import jax
import jax.numpy as jnp
from jax import lax
from jax.experimental import pallas as pl
from jax.experimental.pallas import tpu as pltpu
from jax.experimental.pallas import tpu_sc as plsc

N_NODES = 10000
N_EDGES = 320000
D_IN = 128
D_HID = 16
D_OUT = 128

NTILES = 16
NPT = N_NODES // NTILES
CH = 125
NCHUNKS_TOTAL = N_EDGES // CH
NCH = NCHUNKS_TOTAL // NTILES
EPS = 1e-5


ROWS_BLK = 1000
GRID_ROWS = N_NODES // ROWS_BLK


def _gelu(x):
    return 0.5 * x * (1.0 + lax.erf(x * (2.0 ** -0.5)))


def _tc1_body(x_ref, w1_ref, b1_ref, g1_ref, be1_ref, h_ref):
    x = x_ref[...]
    h = lax.dot_general(x, w1_ref[...], (((1,), (1,)), ((), ())),
                        preferred_element_type=jnp.float32)
    h = _gelu(h + b1_ref[...])
    mu = jnp.mean(h, axis=-1, keepdims=True)
    var = jnp.mean((h - mu) ** 2, axis=-1, keepdims=True)
    h_ref[...] = (h - mu) / jnp.sqrt(var + EPS) * g1_ref[...] + be1_ref[...]


def _tc1(x, W1, b1, g1, be1):
    return pl.pallas_call(
        _tc1_body,
        out_shape=jax.ShapeDtypeStruct((N_NODES, D_HID), jnp.float32),
        grid=(GRID_ROWS,),
        in_specs=[
            pl.BlockSpec((ROWS_BLK, D_IN), lambda i: (i, 0)),
            pl.BlockSpec((D_HID, D_IN), lambda i: (0, 0)),
            pl.BlockSpec((1, D_HID), lambda i: (0, 0)),
            pl.BlockSpec((1, D_HID), lambda i: (0, 0)),
            pl.BlockSpec((1, D_HID), lambda i: (0, 0)),
        ],
        out_specs=pl.BlockSpec((ROWS_BLK, D_HID), lambda i: (i, 0)),
    )(x, W1, b1.reshape(1, D_HID), g1.reshape(1, D_HID), be1.reshape(1, D_HID))


def _rsqrt16(x):
    i = plsc.bitcast(x, jnp.int32)
    i = jnp.int32(0x5F3759DF) - (i >> 1)
    y = plsc.bitcast(i, jnp.float32)
    for _ in range(3):
        y = y * (1.5 - 0.5 * x * y * y)
    return y


def _sc_body(rows_hbm, cols_hbm, h_hbm, taps_hbm,
             u_sh, s_sh, rowix, colix, gbuf0, gbuf1, gbuf2, gbuf3, onesb,
             nbufS, nbufU, d2b, sdb, zbuf,
             gsem0, gsem1, gsem2, gsem3, ssem0, ssem1, ssem2, ssem3):
    cid = lax.axis_index("c")
    tid = lax.axis_index("s")

    @pl.when(cid == 0)
    def _work():
        nbase = tid * NPT
        cbase = tid * NCH

        pltpu.sync_copy(rows_hbm.at[pl.ds(cbase, NCH), :], rowix)
        pltpu.sync_copy(cols_hbm.at[pl.ds(cbase, NCH), :], colix)

        gbufs = (gbuf0, gbuf1, gbuf2, gbuf3)
        gsems = (gsem0, gsem1, gsem2, gsem3)
        ssems = (ssem0, ssem1, ssem2, ssem3)

        def g_fire(j, b):
            pltpu.async_copy(u_sh.at[rowix.at[j]], gbufs[b], gsems[b])

        def g_wait(j, b):
            pltpu.make_async_copy(u_sh.at[rowix.at[j]], gbufs[b],
                                  gsems[b]).wait()

        def s_fire(j, b, src=None):
            pltpu.async_copy(src if src is not None else gbufs[b],
                             s_sh.at[colix.at[j]], ssems[b], add=True)

        def s_wait(j, b, src=None):
            pltpu.make_async_copy(src if src is not None else gbufs[b],
                                  s_sh.at[colix.at[j]], ssems[b]).wait()

        def _fill_const(i, c):
            zbuf[i, :] = jnp.zeros((D_HID,), jnp.float32)
            return c
        lax.fori_loop(0, NPT, _fill_const, 0)

        def _fill_ones(i, c):
            onesb[i, :] = jnp.ones((D_HID,), jnp.float32)
            return c
        lax.fori_loop(0, CH, _fill_ones, 0)

        pltpu.sync_copy(zbuf, s_sh.at[pl.ds(nbase, NPT), :])
        plsc.subcore_barrier()

        for b in range(4):
            s_fire(b, b, src=onesb)

        def _hist(jj, c):
            for b in range(4):
                j = 4 * jj + 4 + b
                s_wait(j - 4, b, src=onesb)
                s_fire(j, b, src=onesb)
            return c
        lax.fori_loop(0, (NCH - 4) // 4, _hist, 0)
        for b in range(4):
            s_wait(NCH - 4 + b, b, src=onesb)
        plsc.subcore_barrier()

        pltpu.sync_copy(s_sh.at[pl.ds(nbase, NPT), :], nbufS)
        pltpu.sync_copy(h_hbm.at[pl.ds(nbase, NPT), :], nbufU)

        def _setup(i, c):
            deg = nbufS[i, :] + 1.0
            r = _rsqrt16(deg)
            d2b[i, :] = 1.0 / deg
            sdb[i, :] = deg * r
            nbufU[i, :] = r * nbufU[i, :]
            return c
        lax.fori_loop(0, NPT, _setup, 0)

        pltpu.sync_copy(nbufU, u_sh.at[pl.ds(nbase, NPT), :])
        pltpu.sync_copy(zbuf, s_sh.at[pl.ds(nbase, NPT), :])
        plsc.subcore_barrier()

        def _round(p, carry):
            g_fire(0, 0)
            g_fire(1, 1)
            g_wait(0, 0)
            s_fire(0, 0)
            g_fire(2, 2)
            g_wait(1, 1)
            s_fire(1, 1)
            g_fire(3, 3)

            def _edges(jj, c):
                for k in range(4):
                    j = 4 * jj + 2 + k
                    b = (2 + k) % 4
                    g_wait(j, b)
                    s_fire(j, b)
                    s_wait(j - 2, (b + 2) % 4)
                    g_fire(j + 2, (b + 2) % 4)
                return c
            lax.fori_loop(0, (NCH - 4) // 4, _edges, 0)
            g_wait(NCH - 2, 2)
            s_fire(NCH - 2, 2)
            s_wait(NCH - 4, 0)
            g_wait(NCH - 1, 3)
            s_fire(NCH - 1, 3)
            s_wait(NCH - 3, 1)
            s_wait(NCH - 2, 2)
            s_wait(NCH - 1, 3)
            plsc.subcore_barrier()

            pltpu.sync_copy(s_sh.at[pl.ds(nbase, NPT), :], nbufS)

            def _update(i, c):
                nbufU[i, :] = d2b[i, :] * (nbufS[i, :] + nbufU[i, :])
                return c
            lax.fori_loop(0, NPT, _update, 0)

            pltpu.sync_copy(nbufU, u_sh.at[pl.ds(nbase, NPT), :])
            pltpu.sync_copy(zbuf, s_sh.at[pl.ds(nbase, NPT), :])

            @pl.when(jnp.logical_and(p >= 6, lax.rem(p, 2) == 0))
            def _():
                def _tap(i, c):
                    nbufS[i, :] = sdb[i, :] * nbufU[i, :]
                    return c
                lax.fori_loop(0, NPT, _tap, 0)
                t = lax.div(p - 6, 2)
                pltpu.sync_copy(nbufS, taps_hbm.at[t, pl.ds(nbase, NPT), :])
            plsc.subcore_barrier()
            return carry
        lax.fori_loop(1, 11, _round, 0)


def _sc_prop(rows_r, cols_r, h):
    mesh = plsc.VectorSubcoreMesh(core_axis_name="c", subcore_axis_name="s")
    f = pl.kernel(
        _sc_body,
        out_type=jax.ShapeDtypeStruct((3, N_NODES, D_HID), jnp.float32),
        mesh=mesh,
        compiler_params=pltpu.CompilerParams(use_tc_tiling_on_sc=False,
                                              needs_layout_passes=False),
        scratch_types=[
            pltpu.VMEM_SHARED((N_NODES, D_HID), jnp.float32),
            pltpu.VMEM_SHARED((N_NODES, D_HID), jnp.float32),
            pltpu.VMEM((NCH, CH), jnp.int32),
            pltpu.VMEM((NCH, CH), jnp.int32),
            pltpu.VMEM((CH, D_HID), jnp.float32),
            pltpu.VMEM((CH, D_HID), jnp.float32),
            pltpu.VMEM((CH, D_HID), jnp.float32),
            pltpu.VMEM((CH, D_HID), jnp.float32),
            pltpu.VMEM((CH, D_HID), jnp.float32),
            pltpu.VMEM((NPT, D_HID), jnp.float32),
            pltpu.VMEM((NPT, D_HID), jnp.float32),
            pltpu.VMEM((NPT, D_HID), jnp.float32),
            pltpu.VMEM((NPT, D_HID), jnp.float32),
            pltpu.VMEM((NPT, D_HID), jnp.float32),
        ] + [pltpu.SemaphoreType.DMA] * 8,
    )
    taps = f(rows_r, cols_r, h)
    return taps[0], taps[1], taps[2]


DH2 = D_HID // 2
NPAD = 10016
NPT8 = NPAD // NTILES
NPTP = NPT8 // 2
NP2 = NPAD // 2


def _sc_body2(rows_hbm, cols_hbm, h2_hbm, taps_hbm,
              u_sh, s_sh, rowix, colix, gbuf0, gbuf1, gbuf2, gbuf3, onesb,
              sbuf, wbuf, zbuf, ubuf, d2b, sdb, tbuf,
              gsem0, gsem1, gsem2, gsem3, ssem0, ssem1, ssem2, ssem3):
    cid = lax.axis_index("c")
    tid = lax.axis_index("s")

    nbase = tid * NPT8
    pbase = tid * NPTP
    cbase = tid * NCH

    pltpu.sync_copy(rows_hbm.at[pl.ds(cbase, NCH), :], rowix)
    pltpu.sync_copy(cols_hbm.at[pl.ds(cbase, NCH), :], colix)

    gbufs = (gbuf0, gbuf1, gbuf2, gbuf3)
    gsems = (gsem0, gsem1, gsem2, gsem3)
    ssems = (ssem0, ssem1, ssem2, ssem3)

    def g_fire(j, b):
        pltpu.async_copy(u_sh.at[rowix.at[j]], gbufs[b], gsems[b])

    def g_wait(j, b):
        pltpu.make_async_copy(u_sh.at[rowix.at[j]], gbufs[b], gsems[b]).wait()

    def s_fire(j, b, src=None):
        pltpu.async_copy(src if src is not None else gbufs[b],
                         s_sh.at[colix.at[j]], ssems[b], add=True)

    def s_wait(j, b, src=None):
        pltpu.make_async_copy(src if src is not None else gbufs[b],
                              s_sh.at[colix.at[j]], ssems[b]).wait()

    i16 = lax.iota(jnp.int32, 16)
    rsel = i16 >> 3
    csel = i16 & 7
    zeros16 = jnp.zeros((16,), jnp.float32)
    ones16 = jnp.ones((16,), jnp.float32)

    def _fill_z(q, c):
        plsc.store_scatter(zbuf, [2 * q + rsel, csel], zeros16)
        return c
    lax.fori_loop(0, NPT8 // 2, _fill_z, 0)

    def _fill_o(q, c):
        plsc.store_scatter(onesb, [2 * q + rsel, csel], ones16)
        return c
    lax.fori_loop(0, (CH + 1) // 2, _fill_o, 0)

    pltpu.sync_copy(zbuf, s_sh.at[pl.ds(nbase, NPT8), :])
    plsc.subcore_barrier()

    for b in range(4):
        s_fire(b, b, src=onesb.at[pl.ds(0, CH), :])

    def _hist(jj, c):
        for b in range(4):
            j = 4 * jj + 4 + b
            s_wait(j - 4, b, src=onesb.at[pl.ds(0, CH), :])
            s_fire(j, b, src=onesb.at[pl.ds(0, CH), :])
        return c
    lax.fori_loop(0, (NCH - 4) // 4, _hist, 0)
    for b in range(4):
        s_wait(NCH - 4 + b, b, src=onesb.at[pl.ds(0, CH), :])
    plsc.subcore_barrier()

    pltpu.sync_copy(s_sh.at[pl.ds(nbase, NPT8), :], sbuf)
    pltpu.sync_copy(h2_hbm.at[cid, pl.ds(pbase, NPTP), :], ubuf)

    def _setup(q, c):
        ridx = 2 * q + rsel
        deg = plsc.load_gather(sbuf, [ridx, csel]) + 1.0
        r = _rsqrt16(deg)
        d2b[q, :] = 1.0 / deg
        sdb[q, :] = deg * r
        u0 = r * ubuf[q, :]
        ubuf[q, :] = u0
        plsc.store_scatter(wbuf, [ridx, csel], u0)
        return c
    lax.fori_loop(0, NPTP, _setup, 0)

    pltpu.sync_copy(wbuf, u_sh.at[pl.ds(nbase, NPT8), :])
    pltpu.sync_copy(zbuf, s_sh.at[pl.ds(nbase, NPT8), :])
    plsc.subcore_barrier()

    def _round(p, carry):
        g_fire(0, 0)
        g_fire(1, 1)
        g_wait(0, 0)
        s_fire(0, 0)
        g_fire(2, 2)
        g_wait(1, 1)
        s_fire(1, 1)
        g_fire(3, 3)

        def _edges(jj, c):
            for k in range(4):
                j = 4 * jj + 2 + k
                b = (2 + k) % 4
                g_wait(j, b)
                s_fire(j, b)
                s_wait(j - 2, (b + 2) % 4)
                g_fire(j + 2, (b + 2) % 4)
            return c
        lax.fori_loop(0, (NCH - 4) // 4, _edges, 0)
        g_wait(NCH - 2, 2)
        s_fire(NCH - 2, 2)
        s_wait(NCH - 4, 0)
        g_wait(NCH - 1, 3)
        s_fire(NCH - 1, 3)
        s_wait(NCH - 3, 1)
        s_wait(NCH - 2, 2)
        s_wait(NCH - 1, 3)
        plsc.subcore_barrier()

        pltpu.sync_copy(s_sh.at[pl.ds(nbase, NPT8), :], sbuf)

        def _update(q, c):
            ridx = 2 * q + rsel
            sv = plsc.load_gather(sbuf, [ridx, csel])
            un = d2b[q, :] * (sv + ubuf[q, :])
            ubuf[q, :] = un
            plsc.store_scatter(wbuf, [ridx, csel], un)
            return c
        lax.fori_loop(0, NPTP, _update, 0)

        pltpu.sync_copy(wbuf, u_sh.at[pl.ds(nbase, NPT8), :])
        pltpu.sync_copy(zbuf, s_sh.at[pl.ds(nbase, NPT8), :])

        @pl.when(jnp.logical_and(p >= 6, lax.rem(p, 2) == 0))
        def _():
            def _tap(q, c):
                tbuf[q, :] = sdb[q, :] * ubuf[q, :]
                return c
            lax.fori_loop(0, NPTP, _tap, 0)
            t = lax.div(p - 6, 2)
            pltpu.sync_copy(tbuf, taps_hbm.at[t, cid, pl.ds(pbase, NPTP), :])
        plsc.subcore_barrier()
        return carry
    lax.fori_loop(1, 11, _round, 0)


def _sc_prop2(rows_r, cols_r, h2):
    mesh = plsc.VectorSubcoreMesh(core_axis_name="c", subcore_axis_name="s")
    f = pl.kernel(
        _sc_body2,
        out_type=jax.ShapeDtypeStruct((3, 2, NP2, D_HID), jnp.float32),
        mesh=mesh,
        compiler_params=pltpu.CompilerParams(use_tc_tiling_on_sc=False,
                                             needs_layout_passes=False),
        scratch_types=[
            pltpu.VMEM_SHARED((NPAD, DH2), jnp.float32),
            pltpu.VMEM_SHARED((NPAD, DH2), jnp.float32),
            pltpu.VMEM((NCH, CH), jnp.int32),
            pltpu.VMEM((NCH, CH), jnp.int32),
            pltpu.VMEM((CH, DH2), jnp.float32),
            pltpu.VMEM((CH, DH2), jnp.float32),
            pltpu.VMEM((CH, DH2), jnp.float32),
            pltpu.VMEM((CH, DH2), jnp.float32),
            pltpu.VMEM((CH + 1, DH2), jnp.float32),
            pltpu.VMEM((NPT8, DH2), jnp.float32),
            pltpu.VMEM((NPT8, DH2), jnp.float32),
            pltpu.VMEM((NPT8, DH2), jnp.float32),
            pltpu.VMEM((NPTP, D_HID), jnp.float32),
            pltpu.VMEM((NPTP, D_HID), jnp.float32),
            pltpu.VMEM((NPTP, D_HID), jnp.float32),
            pltpu.VMEM((NPTP, D_HID), jnp.float32),
        ] + [pltpu.SemaphoreType.DMA] * 8,
    )
    return f(rows_r, cols_r, h2)


D_CAT = 3 * D_HID


def _tc2_body(c6_ref, c8_ref, c10_ref, w6_ref, b6_ref, w8_ref, b8_ref,
              w10_ref, b10_ref, g2_ref, be2_ref, w2_ref, b2_ref, out_ref):
    def lin(c_ref, w_ref, b_ref):
        return lax.dot_general(c_ref[...], w_ref[...], (((1,), (1,)), ((), ())),
                               preferred_element_type=jnp.float32) + b_ref[...]

    t6 = _gelu(lin(c6_ref, w6_ref, b6_ref))
    t8 = _gelu(lin(c8_ref, w8_ref, b8_ref))
    t10 = _gelu(lin(c10_ref, w10_ref, b10_ref))

    s = jnp.sum(t6, axis=-1, keepdims=True) + jnp.sum(t8, axis=-1, keepdims=True) \
        + jnp.sum(t10, axis=-1, keepdims=True)
    mu = s / D_CAT
    v = (jnp.sum((t6 - mu) ** 2, axis=-1, keepdims=True)
         + jnp.sum((t8 - mu) ** 2, axis=-1, keepdims=True)
         + jnp.sum((t10 - mu) ** 2, axis=-1, keepdims=True)) / D_CAT
    inv = 1.0 / jnp.sqrt(v + EPS)

    g2 = g2_ref[...]
    be2 = be2_ref[...]
    w2 = w2_ref[...]
    acc = jnp.zeros_like(out_ref[...]) + b2_ref[...]
    for k, t in enumerate((t6, t8, t10)):
        nk = (t - mu) * inv * g2[:, k * D_HID:(k + 1) * D_HID] \
            + be2[:, k * D_HID:(k + 1) * D_HID]
        acc = acc + lax.dot_general(
            nk, w2[:, k * D_HID:(k + 1) * D_HID], (((1,), (1,)), ((), ())),
            preferred_element_type=jnp.float32)
    out_ref[...] = acc


def _tc2(c6, c8, c10, W6, b6, W8, b8, W10, b10, g2, be2, W2, b2):
    blk16 = pl.BlockSpec((ROWS_BLK, D_HID), lambda i: (i, 0))
    w16 = pl.BlockSpec((D_HID, D_HID), lambda i: (0, 0))
    v16 = pl.BlockSpec((1, D_HID), lambda i: (0, 0))
    v48 = pl.BlockSpec((1, D_CAT), lambda i: (0, 0))
    return pl.pallas_call(
        _tc2_body,
        out_shape=jax.ShapeDtypeStruct((N_NODES, D_OUT), jnp.float32),
        grid=(GRID_ROWS,),
        in_specs=[
            blk16, blk16, blk16,
            w16, v16, w16, v16, w16, v16,
            v48, v48,
            pl.BlockSpec((D_OUT, D_CAT), lambda i: (0, 0)),
            pl.BlockSpec((1, D_OUT), lambda i: (0, 0)),
        ],
        out_specs=pl.BlockSpec((ROWS_BLK, D_OUT), lambda i: (i, 0)),
    )(c6, c8, c10,
      W6, b6.reshape(1, D_HID), W8, b8.reshape(1, D_HID),
      W10, b10.reshape(1, D_HID),
      g2.reshape(1, D_CAT), be2.reshape(1, D_CAT),
      W2, b2.reshape(1, D_OUT))


def kernel(x, edge_index, W1, b1, W6, b6, W8, b8, W10, b10,
           g1, be1, g2, be2, W2, b2):
    h = _tc1(x, W1, b1, g1, be1)
    rows_r = edge_index[0].reshape(NCHUNKS_TOTAL, CH)
    cols_r = edge_index[1].reshape(NCHUNKS_TOTAL, CH)
    hp = jnp.pad(h, ((0, NPAD - N_NODES), (0, 0)))
    h2 = jnp.stack([hp[:, :DH2].reshape(NP2, D_HID),
                    hp[:, DH2:].reshape(NP2, D_HID)])
    taps = _sc_prop2(rows_r, cols_r, h2).reshape(3, 2, NPAD, DH2)
    c6 = jnp.concatenate([taps[0, 0], taps[0, 1]], axis=1)[:N_NODES]
    c8 = jnp.concatenate([taps[1, 0], taps[1, 1]], axis=1)[:N_NODES]
    c10 = jnp.concatenate([taps[2, 0], taps[2, 1]], axis=1)[:N_NODES]
    return _tc2(c6, c8, c10, W6, b6, W8, b8, W10, b10, g2, be2, W2, b2)


# speedup vs baseline: 2.7241x; 1.1612x over previous
import jax
import jax.numpy as jnp
from jax import lax
from jax.experimental import pallas as pl
from jax.experimental.pallas import tpu as pltpu
from jax.experimental.pallas import tpu_sc as plsc

N_NODES = 10000
N_EDGES = 320000
D_IN = 128
D_HID = 16
D_OUT = 128

NTILES = 16
TCB = 2504
NPT = N_NODES // NTILES
CH = 125
NCHUNKS_TOTAL = N_EDGES // CH
NCH = NCHUNKS_TOTAL // NTILES
EPS = 1e-5


def _gelu(x):
    return 0.5 * x * (1.0 + lax.erf(x * (2.0 ** -0.5)))


def _ln16(h, g, be):
    mu = jnp.mean(h, axis=-1, keepdims=True)
    var = jnp.mean((h - mu) ** 2, axis=-1, keepdims=True)
    return (h - mu) / jnp.sqrt(var + EPS) * g + be


def _tc1_body(xa_ref, xb_ref, w1_ref, b1_ref, g1_ref, be1_ref, h2_ref):
    def feat(x_ref):
        h = lax.dot_general(x_ref[...], w1_ref[...], (((1,), (1,)), ((), ())),
                            preferred_element_type=jnp.float32)
        return _ln16(_gelu(h + b1_ref[...]), g1_ref[...], be1_ref[...])

    ha = feat(xa_ref)
    hb = feat(xb_ref)
    h2_ref[0] = jnp.concatenate([ha[:, :8], hb[:, :8]], axis=-1)
    h2_ref[1] = jnp.concatenate([ha[:, 8:], hb[:, 8:]], axis=-1)


def _tc1(x, W1, b1, g1, be1):
    xblk = pl.BlockSpec((TCB, D_IN), lambda i: (i, 0))
    xblk_b = pl.BlockSpec((TCB, D_IN), lambda i: (i + 2, 0))
    v16 = pl.BlockSpec((1, D_HID), lambda i: (0, 0))
    return pl.pallas_call(
        _tc1_body,
        out_shape=jax.ShapeDtypeStruct((2, NP2, D_HID), jnp.float32),
        grid=(2,),
        in_specs=[
            xblk, xblk_b,
            pl.BlockSpec((D_HID, D_IN), lambda i: (0, 0)),
            v16, v16, v16,
        ],
        out_specs=pl.BlockSpec((2, TCB, D_HID), lambda i: (0, i, 0)),
    )(x, x, W1, b1.reshape(1, D_HID), g1.reshape(1, D_HID),
      be1.reshape(1, D_HID))


def _rsqrt16(x):
    i = plsc.bitcast(x, jnp.int32)
    i = jnp.int32(0x5F3759DF) - (i >> 1)
    y = plsc.bitcast(i, jnp.float32)
    for _ in range(3):
        y = y * (1.5 - 0.5 * x * y * y)
    return y


def _sc_body(rows_hbm, cols_hbm, h_hbm, taps_hbm,
             u_sh, s_sh, rowix, colix, gbuf0, gbuf1, gbuf2, gbuf3, onesb,
             nbufS, nbufU, d2b, sdb, zbuf,
             gsem0, gsem1, gsem2, gsem3, ssem0, ssem1, ssem2, ssem3):
    cid = lax.axis_index("c")
    tid = lax.axis_index("s")

    @pl.when(cid == 0)
    def _work():
        nbase = tid * NPT
        cbase = tid * NCH

        pltpu.sync_copy(rows_hbm.at[pl.ds(cbase, NCH), :], rowix)
        pltpu.sync_copy(cols_hbm.at[pl.ds(cbase, NCH), :], colix)

        gbufs = (gbuf0, gbuf1, gbuf2, gbuf3)
        gsems = (gsem0, gsem1, gsem2, gsem3)
        ssems = (ssem0, ssem1, ssem2, ssem3)

        def g_fire(j, b):
            pltpu.async_copy(u_sh.at[rowix.at[j]], gbufs[b], gsems[b])

        def g_wait(j, b):
            pltpu.make_async_copy(u_sh.at[rowix.at[j]], gbufs[b],
                                  gsems[b]).wait()

        def s_fire(j, b, src=None):
            pltpu.async_copy(src if src is not None else gbufs[b],
                             s_sh.at[colix.at[j]], ssems[b], add=True)

        def s_wait(j, b, src=None):
            pltpu.make_async_copy(src if src is not None else gbufs[b],
                                  s_sh.at[colix.at[j]], ssems[b]).wait()

        def _fill_const(i, c):
            zbuf[i, :] = jnp.zeros((D_HID,), jnp.float32)
            return c
        lax.fori_loop(0, NPT, _fill_const, 0)

        def _fill_ones(i, c):
            onesb[i, :] = jnp.ones((D_HID,), jnp.float32)
            return c
        lax.fori_loop(0, CH, _fill_ones, 0)

        pltpu.sync_copy(zbuf, s_sh.at[pl.ds(nbase, NPT), :])
        plsc.subcore_barrier()

        for b in range(4):
            s_fire(b, b, src=onesb)

        def _hist(jj, c):
            for b in range(4):
                j = 4 * jj + 4 + b
                s_wait(j - 4, b, src=onesb)
                s_fire(j, b, src=onesb)
            return c
        lax.fori_loop(0, (NCH - 4) // 4, _hist, 0)
        for b in range(4):
            s_wait(NCH - 4 + b, b, src=onesb)
        plsc.subcore_barrier()

        pltpu.sync_copy(s_sh.at[pl.ds(nbase, NPT), :], nbufS)
        pltpu.sync_copy(h_hbm.at[pl.ds(nbase, NPT), :], nbufU)

        def _setup(i, c):
            deg = nbufS[i, :] + 1.0
            r = _rsqrt16(deg)
            d2b[i, :] = 1.0 / deg
            sdb[i, :] = deg * r
            nbufU[i, :] = r * nbufU[i, :]
            return c
        lax.fori_loop(0, NPT, _setup, 0)

        pltpu.sync_copy(nbufU, u_sh.at[pl.ds(nbase, NPT), :])
        pltpu.sync_copy(zbuf, s_sh.at[pl.ds(nbase, NPT), :])
        plsc.subcore_barrier()

        def _round(p, carry):
            g_fire(0, 0)
            g_fire(1, 1)
            g_wait(0, 0)
            s_fire(0, 0)
            g_fire(2, 2)
            g_wait(1, 1)
            s_fire(1, 1)
            g_fire(3, 3)

            def _edges(jj, c):
                for k in range(4):
                    j = 4 * jj + 2 + k
                    b = (2 + k) % 4
                    g_wait(j, b)
                    s_fire(j, b)
                    s_wait(j - 2, (b + 2) % 4)
                    g_fire(j + 2, (b + 2) % 4)
                return c
            lax.fori_loop(0, (NCH - 4) // 4, _edges, 0)
            g_wait(NCH - 2, 2)
            s_fire(NCH - 2, 2)
            s_wait(NCH - 4, 0)
            g_wait(NCH - 1, 3)
            s_fire(NCH - 1, 3)
            s_wait(NCH - 3, 1)
            s_wait(NCH - 2, 2)
            s_wait(NCH - 1, 3)
            plsc.subcore_barrier()

            pltpu.sync_copy(s_sh.at[pl.ds(nbase, NPT), :], nbufS)

            def _update(i, c):
                nbufU[i, :] = d2b[i, :] * (nbufS[i, :] + nbufU[i, :])
                return c
            lax.fori_loop(0, NPT, _update, 0)

            pltpu.sync_copy(nbufU, u_sh.at[pl.ds(nbase, NPT), :])
            pltpu.sync_copy(zbuf, s_sh.at[pl.ds(nbase, NPT), :])

            @pl.when(jnp.logical_and(p >= 6, lax.rem(p, 2) == 0))
            def _():
                def _tap(i, c):
                    nbufS[i, :] = sdb[i, :] * nbufU[i, :]
                    return c
                lax.fori_loop(0, NPT, _tap, 0)
                t = lax.div(p - 6, 2)
                pltpu.sync_copy(nbufS, taps_hbm.at[t, pl.ds(nbase, NPT), :])
            plsc.subcore_barrier()
            return carry
        lax.fori_loop(1, 11, _round, 0)


def _sc_prop(rows_r, cols_r, h):
    mesh = plsc.VectorSubcoreMesh(core_axis_name="c", subcore_axis_name="s")
    f = pl.kernel(
        _sc_body,
        out_type=jax.ShapeDtypeStruct((3, N_NODES, D_HID), jnp.float32),
        mesh=mesh,
        compiler_params=pltpu.CompilerParams(use_tc_tiling_on_sc=False,
                                              needs_layout_passes=False),
        scratch_types=[
            pltpu.VMEM_SHARED((N_NODES, D_HID), jnp.float32),
            pltpu.VMEM_SHARED((N_NODES, D_HID), jnp.float32),
            pltpu.VMEM((NCH, CH), jnp.int32),
            pltpu.VMEM((NCH, CH), jnp.int32),
            pltpu.VMEM((CH, D_HID), jnp.float32),
            pltpu.VMEM((CH, D_HID), jnp.float32),
            pltpu.VMEM((CH, D_HID), jnp.float32),
            pltpu.VMEM((CH, D_HID), jnp.float32),
            pltpu.VMEM((CH, D_HID), jnp.float32),
            pltpu.VMEM((NPT, D_HID), jnp.float32),
            pltpu.VMEM((NPT, D_HID), jnp.float32),
            pltpu.VMEM((NPT, D_HID), jnp.float32),
            pltpu.VMEM((NPT, D_HID), jnp.float32),
            pltpu.VMEM((NPT, D_HID), jnp.float32),
        ] + [pltpu.SemaphoreType.DMA] * 8,
    )
    taps = f(rows_r, cols_r, h)
    return taps[0], taps[1], taps[2]


DH2 = D_HID // 2
NPAD = 10016
NPT8 = NPAD // NTILES
NPTP = NPT8 // 2
NP2 = NPAD // 2


def _sc_body2(rows_hbm, cols_hbm, h2_hbm, taps_hbm,
              u_sh, s_sh, rowix, colix, gbuf0, gbuf1, gbuf2, gbuf3, onesb,
              sbuf, wbuf, zbuf, ubuf, d2b, sdb, tbuf,
              gsem0, gsem1, gsem2, gsem3, ssem0, ssem1, ssem2, ssem3):
    cid = lax.axis_index("c")
    tid = lax.axis_index("s")

    pbase = tid * NPTP
    bbase = NP2 + pbase
    cbase = tid * NCH

    def node_write(buf, arr):
        pltpu.sync_copy(buf.at[pl.ds(0, NPTP), :], arr.at[pl.ds(pbase, NPTP), :])
        pltpu.sync_copy(buf.at[pl.ds(NPTP, NPTP), :], arr.at[pl.ds(bbase, NPTP), :])

    def node_read(arr, buf):
        pltpu.sync_copy(arr.at[pl.ds(pbase, NPTP), :], buf.at[pl.ds(0, NPTP), :])
        pltpu.sync_copy(arr.at[pl.ds(bbase, NPTP), :], buf.at[pl.ds(NPTP, NPTP), :])

    pltpu.sync_copy(rows_hbm.at[pl.ds(cbase, NCH), :], rowix)
    pltpu.sync_copy(cols_hbm.at[pl.ds(cbase, NCH), :], colix)

    gbufs = (gbuf0, gbuf1, gbuf2, gbuf3)
    gsems = (gsem0, gsem1, gsem2, gsem3)
    ssems = (ssem0, ssem1, ssem2, ssem3)

    def g_fire(j, b):
        pltpu.async_copy(u_sh.at[rowix.at[j]], gbufs[b], gsems[b])

    def g_wait(j, b):
        pltpu.make_async_copy(u_sh.at[rowix.at[j]], gbufs[b], gsems[b]).wait()

    def s_fire(j, b, src=None):
        pltpu.async_copy(src if src is not None else gbufs[b],
                         s_sh.at[colix.at[j]], ssems[b], add=True)

    def s_wait(j, b, src=None):
        pltpu.make_async_copy(src if src is not None else gbufs[b],
                              s_sh.at[colix.at[j]], ssems[b]).wait()

    i16 = lax.iota(jnp.int32, 16)
    rsel = (i16 >> 3) * NPTP
    csel = i16 & 7
    zeros16 = jnp.zeros((16,), jnp.float32)
    ones16 = jnp.ones((16,), jnp.float32)

    def _fill_z(q, c):
        plsc.store_scatter(zbuf, [q + rsel, csel], zeros16)
        return c
    lax.fori_loop(0, NPTP, _fill_z, 0)

    osel = (i16 >> 3) * ((CH + 1) // 2)

    def _fill_o(q, c):
        plsc.store_scatter(onesb, [q + osel, csel], ones16)
        return c
    lax.fori_loop(0, (CH + 1) // 2, _fill_o, 0)

    node_write(zbuf, s_sh)
    plsc.subcore_barrier()

    for b in range(4):
        s_fire(b, b, src=onesb.at[pl.ds(0, CH), :])

    def _hist(jj, c):
        for b in range(4):
            j = 4 * jj + 4 + b
            s_wait(j - 4, b, src=onesb.at[pl.ds(0, CH), :])
            s_fire(j, b, src=onesb.at[pl.ds(0, CH), :])
        return c
    lax.fori_loop(0, (NCH - 4) // 4, _hist, 0)
    for b in range(4):
        s_wait(NCH - 4 + b, b, src=onesb.at[pl.ds(0, CH), :])
    plsc.subcore_barrier()

    node_read(s_sh, sbuf)
    pltpu.sync_copy(h2_hbm.at[cid, pl.ds(pbase, NPTP), :], ubuf)

    def _setup(q, c):
        ridx = q + rsel
        deg = plsc.load_gather(sbuf, [ridx, csel]) + 1.0
        r = _rsqrt16(deg)
        d2b[q, :] = 1.0 / deg
        sdb[q, :] = deg * r
        u0 = r * ubuf[q, :]
        ubuf[q, :] = u0
        plsc.store_scatter(wbuf, [ridx, csel], u0)
        return c
    lax.fori_loop(0, NPTP, _setup, 0)

    node_write(wbuf, u_sh)
    node_write(zbuf, s_sh)
    plsc.subcore_barrier()

    def _round(p, carry):
        g_fire(0, 0)
        g_fire(1, 1)
        g_wait(0, 0)
        s_fire(0, 0)
        g_fire(2, 2)
        g_wait(1, 1)
        s_fire(1, 1)
        g_fire(3, 3)

        def _edges(jj, c):
            for k in range(4):
                j = 4 * jj + 2 + k
                b = (2 + k) % 4
                g_wait(j, b)
                s_fire(j, b)
                s_wait(j - 2, (b + 2) % 4)
                g_fire(j + 2, (b + 2) % 4)
            return c
        lax.fori_loop(0, (NCH - 4) // 4, _edges, 0)
        g_wait(NCH - 2, 2)
        s_fire(NCH - 2, 2)
        s_wait(NCH - 4, 0)
        g_wait(NCH - 1, 3)
        s_fire(NCH - 1, 3)
        s_wait(NCH - 3, 1)
        s_wait(NCH - 2, 2)
        s_wait(NCH - 1, 3)
        plsc.subcore_barrier()

        node_read(s_sh, sbuf)

        def _update(q, c):
            ridx = q + rsel
            sv = plsc.load_gather(sbuf, [ridx, csel])
            un = d2b[q, :] * (sv + ubuf[q, :])
            ubuf[q, :] = un
            plsc.store_scatter(wbuf, [ridx, csel], un)
            return c
        lax.fori_loop(0, NPTP, _update, 0)

        node_write(wbuf, u_sh)
        node_write(zbuf, s_sh)

        @pl.when(jnp.logical_and(p >= 6, lax.rem(p, 2) == 0))
        def _():
            def _tap(q, c):
                tbuf[q, :] = sdb[q, :] * ubuf[q, :]
                return c
            lax.fori_loop(0, NPTP, _tap, 0)
            t = lax.div(p - 6, 2)
            pltpu.sync_copy(tbuf, taps_hbm.at[t, cid, pl.ds(pbase, NPTP), :])
        plsc.subcore_barrier()
        return carry
    lax.fori_loop(1, 11, _round, 0)


def _sc_prop2(rows_r, cols_r, h2):
    mesh = plsc.VectorSubcoreMesh(core_axis_name="c", subcore_axis_name="s")
    f = pl.kernel(
        _sc_body2,
        out_type=jax.ShapeDtypeStruct((3, 2, NP2, D_HID), jnp.float32),
        mesh=mesh,
        compiler_params=pltpu.CompilerParams(use_tc_tiling_on_sc=False,
                                             needs_layout_passes=False),
        scratch_types=[
            pltpu.VMEM_SHARED((NPAD, DH2), jnp.float32),
            pltpu.VMEM_SHARED((NPAD, DH2), jnp.float32),
            pltpu.VMEM((NCH, CH), jnp.int32),
            pltpu.VMEM((NCH, CH), jnp.int32),
            pltpu.VMEM((CH, DH2), jnp.float32),
            pltpu.VMEM((CH, DH2), jnp.float32),
            pltpu.VMEM((CH, DH2), jnp.float32),
            pltpu.VMEM((CH, DH2), jnp.float32),
            pltpu.VMEM((CH + 1, DH2), jnp.float32),
            pltpu.VMEM((NPT8, DH2), jnp.float32),
            pltpu.VMEM((NPT8, DH2), jnp.float32),
            pltpu.VMEM((NPT8, DH2), jnp.float32),
            pltpu.VMEM((NPTP, D_HID), jnp.float32),
            pltpu.VMEM((NPTP, D_HID), jnp.float32),
            pltpu.VMEM((NPTP, D_HID), jnp.float32),
            pltpu.VMEM((NPTP, D_HID), jnp.float32),
        ] + [pltpu.SemaphoreType.DMA] * 8,
    )
    return f(rows_r, cols_r, h2)


D_CAT = 3 * D_HID


def _tc2_body(taps_ref, w6_ref, b6_ref, w8_ref, b8_ref,
              w10_ref, b10_ref, g2_ref, be2_ref, w2_ref, b2_ref, out_ref):
    hi = pl.program_id(0) >= 2

    def unpack(t):
        t0 = taps_ref[t, 0]
        t1 = taps_ref[t, 1]
        ca = jnp.concatenate([t0[:, :DH2], t1[:, :DH2]], axis=-1)
        cb = jnp.concatenate([t0[:, DH2:], t1[:, DH2:]], axis=-1)
        return jnp.where(hi, cb, ca)

    def lin(c, w_ref, b_ref):
        return lax.dot_general(c, w_ref[...], (((1,), (1,)), ((), ())),
                               preferred_element_type=jnp.float32) + b_ref[...]

    t6 = _gelu(lin(unpack(0), w6_ref, b6_ref))
    t8 = _gelu(lin(unpack(1), w8_ref, b8_ref))
    t10 = _gelu(lin(unpack(2), w10_ref, b10_ref))

    s = jnp.sum(t6, axis=-1, keepdims=True) + jnp.sum(t8, axis=-1, keepdims=True) \
        + jnp.sum(t10, axis=-1, keepdims=True)
    mu = s / D_CAT
    v = (jnp.sum((t6 - mu) ** 2, axis=-1, keepdims=True)
         + jnp.sum((t8 - mu) ** 2, axis=-1, keepdims=True)
         + jnp.sum((t10 - mu) ** 2, axis=-1, keepdims=True)) / D_CAT
    inv = 1.0 / jnp.sqrt(v + EPS)

    g2 = g2_ref[...]
    be2 = be2_ref[...]
    w2 = w2_ref[...]
    acc = jnp.zeros_like(out_ref[...]) + b2_ref[...]
    for k, t in enumerate((t6, t8, t10)):
        nk = (t - mu) * inv * g2[:, k * D_HID:(k + 1) * D_HID] \
            + be2[:, k * D_HID:(k + 1) * D_HID]
        acc = acc + lax.dot_general(
            nk, w2[:, k * D_HID:(k + 1) * D_HID], (((1,), (1,)), ((), ())),
            preferred_element_type=jnp.float32)
    out_ref[...] = acc


def _tc2(taps, W6, b6, W8, b8, W10, b10, g2, be2, W2, b2):
    w16 = pl.BlockSpec((D_HID, D_HID), lambda i: (0, 0))
    v16 = pl.BlockSpec((1, D_HID), lambda i: (0, 0))
    v48 = pl.BlockSpec((1, D_CAT), lambda i: (0, 0))
    return pl.pallas_call(
        _tc2_body,
        out_shape=jax.ShapeDtypeStruct((N_NODES, D_OUT), jnp.float32),
        grid=(4,),
        in_specs=[
            pl.BlockSpec((3, 2, TCB, D_HID),
                         lambda i: (0, 0, lax.rem(i, 2), 0)),
            w16, v16, w16, v16, w16, v16,
            v48, v48,
            pl.BlockSpec((D_OUT, D_CAT), lambda i: (0, 0)),
            pl.BlockSpec((1, D_OUT), lambda i: (0, 0)),
        ],
        out_specs=pl.BlockSpec((TCB, D_OUT), lambda i: (i, 0)),
    )(taps,
      W6, b6.reshape(1, D_HID), W8, b8.reshape(1, D_HID),
      W10, b10.reshape(1, D_HID),
      g2.reshape(1, D_CAT), be2.reshape(1, D_CAT),
      W2, b2.reshape(1, D_OUT))


def kernel(x, edge_index, W1, b1, W6, b6, W8, b8, W10, b10,
           g1, be1, g2, be2, W2, b2):
    h2 = _tc1(x, W1, b1, g1, be1)
    rows_r = edge_index[0].reshape(NCHUNKS_TOTAL, CH)
    cols_r = edge_index[1].reshape(NCHUNKS_TOTAL, CH)
    taps = _sc_prop2(rows_r, cols_r, h2)
    return _tc2(taps, W6, b6, W8, b8, W10, b10, g2, be2, W2, b2)
